# baseline (speedup 1.0000x reference)
import functools

import jax
import jax.numpy as jnp
from jax import lax
from jax.experimental import pallas as pl
from jax.experimental.pallas import tpu as pltpu
from jax.experimental.pallas import tpu_sc as plsc

N_TOKENS = 771
DIM = 512
NUM_SPATIAL = 256

STRIDE = 24
ROWS = 32
NSLOT = 3
LANES = 16


def _bias_body(te_ref, sp_ref, gl_ref, out_ref):
    te = te_ref[...]
    sp = sp_ref[0]
    gl = gl_ref[0]
    bias = jnp.concatenate(
        [
            te[0:1] + gl[0:1],
            te[1:2] + gl[1:2],
            te[2:3] + sp,
            te[3:4] + sp,
            te[4:5] + gl[2:3],
            te[5:6] + sp,
        ],
        axis=0,
    )
    out_ref[...] = bias


def _sc_body(tok_hbm, bias_hbm, out_hbm, bias_buf, tok_buf, in_sems, out_sems):
    B = tok_hbm.shape[0]
    num_cores = plsc.get_sparse_core_info().num_cores
    wid = lax.axis_index("s") * num_cores + lax.axis_index("c")
    start = wid * STRIDE

    pltpu.sync_copy(bias_hbm.at[pl.ds(start, ROWS)], bias_buf)

    def in_copy(b, slot):
        return pltpu.make_async_copy(
            tok_hbm.at[b, pl.ds(start, ROWS)], tok_buf.at[slot], in_sems.at[slot]
        )

    def out_copy(b, slot):
        return pltpu.make_async_copy(
            tok_buf.at[slot], out_hbm.at[b, pl.ds(start, ROWS)], out_sems.at[slot]
        )

    for p in range(NSLOT):
        in_copy(p, p).start()

    def step(b, _):
        slot = lax.rem(b, NSLOT)
        in_copy(b, slot).wait()
        for r in range(ROWS):
            for j in range(DIM // LANES):
                breg = bias_buf[r, pl.ds(j * LANES, LANES)]
                plsc.addupdate(tok_buf.at[slot, r, pl.ds(j * LANES, LANES)], breg)
        out_copy(b, slot).start()

        @pl.when(jnp.logical_and(b >= 1, b - 1 + NSLOT < B))
        def _():
            prev = lax.rem(b - 1, NSLOT)
            out_copy(b - 1, prev).wait()
            in_copy(b - 1 + NSLOT, prev).start()

        return 0

    lax.fori_loop(0, B, step, 0)

    for t in range(B - NSLOT, B):
        out_copy(t, t % NSLOT).wait()


def kernel(tokens, type_embedding, spatial_pos_embedding, global_pos_embedding):
    B, N, D = tokens.shape

    bias = pl.pallas_call(
        _bias_body,
        out_shape=jax.ShapeDtypeStruct((N, D), tokens.dtype),
    )(type_embedding, spatial_pos_embedding, global_pos_embedding)

    mesh = plsc.VectorSubcoreMesh(core_axis_name="c", subcore_axis_name="s")
    sc_add = functools.partial(
        pl.kernel,
        mesh=mesh,
        out_type=jax.ShapeDtypeStruct((B, N, D), tokens.dtype),
        scratch_types=[
            pltpu.VMEM((ROWS, D), tokens.dtype),
            pltpu.VMEM((NSLOT, ROWS, D), tokens.dtype),
            pltpu.SemaphoreType.DMA((NSLOT,)),
            pltpu.SemaphoreType.DMA((NSLOT,)),
        ],
    )(_sc_body)
    return sc_add(tokens, bias)


# speedup vs baseline: 1.5316x; 1.5316x over previous
import functools

import jax
import jax.numpy as jnp
from jax import lax
from jax.experimental import pallas as pl
from jax.experimental.pallas import tpu as pltpu
from jax.experimental.pallas import tpu_sc as plsc

N_TOKENS = 771
DIM = 512
NUM_SPATIAL = 256

STRIDE = 24
ROWS = 32
NSLOT = 3
LANES = 16


def _bias_body(te_ref, sp_ref, gl_ref, out_ref):
    te = te_ref[...]
    sp = sp_ref[0]
    gl = gl_ref[0]
    bias = jnp.concatenate(
        [
            te[0:1] + gl[0:1],
            te[1:2] + gl[1:2],
            te[2:3] + sp,
            te[3:4] + sp,
            te[4:5] + gl[2:3],
            te[5:6] + sp,
        ],
        axis=0,
    )
    out_ref[...] = bias


def _sc_body(tok_hbm, bias_hbm, out_hbm, bias_buf, tok_buf, in_sems, out_sems):
    B = tok_hbm.shape[0]
    num_cores = plsc.get_sparse_core_info().num_cores
    wid = lax.axis_index("s") * num_cores + lax.axis_index("c")
    start = wid * STRIDE

    pltpu.sync_copy(bias_hbm.at[pl.ds(start, ROWS)], bias_buf)

    def in_copy(b, slot):
        return pltpu.make_async_copy(
            tok_hbm.at[b, pl.ds(start, ROWS)], tok_buf.at[slot], in_sems.at[slot]
        )

    def out_copy(b, slot):
        return pltpu.make_async_copy(
            tok_buf.at[slot], out_hbm.at[b, pl.ds(start, ROWS)], out_sems.at[slot]
        )

    for p in range(NSLOT):
        in_copy(p, p).start()

    def step(b, _):
        slot = lax.rem(b, NSLOT)
        in_copy(b, slot).wait()

        @plsc.parallel_loop(0, ROWS, unroll=2)
        def _(r):
            for j in range(DIM // LANES):
                breg = bias_buf[r, pl.ds(j * LANES, LANES)]
                plsc.addupdate(tok_buf.at[slot, r, pl.ds(j * LANES, LANES)], breg)

        out_copy(b, slot).start()

        @pl.when(jnp.logical_and(b >= 1, b - 1 + NSLOT < B))
        def _():
            prev = lax.rem(b - 1, NSLOT)
            out_copy(b - 1, prev).wait()
            in_copy(b - 1 + NSLOT, prev).start()

        return 0

    lax.fori_loop(0, B, step, 0)

    for t in range(B - NSLOT, B):
        out_copy(t, t % NSLOT).wait()


def kernel(tokens, type_embedding, spatial_pos_embedding, global_pos_embedding):
    B, N, D = tokens.shape

    bias = pl.pallas_call(
        _bias_body,
        out_shape=jax.ShapeDtypeStruct((N, D), tokens.dtype),
    )(type_embedding, spatial_pos_embedding, global_pos_embedding)

    mesh = plsc.VectorSubcoreMesh(
        core_axis_name="c", subcore_axis_name="s", num_cores=2
    )
    sc_add = functools.partial(
        pl.kernel,
        mesh=mesh,
        out_type=jax.ShapeDtypeStruct((B, N, D), tokens.dtype),
        scratch_types=[
            pltpu.VMEM((ROWS, D), tokens.dtype),
            pltpu.VMEM((NSLOT, ROWS, D), tokens.dtype),
            pltpu.SemaphoreType.DMA((NSLOT,)),
            pltpu.SemaphoreType.DMA((NSLOT,)),
        ],
    )(_sc_body)
    return sc_add(tokens, bias)
